# baseline (speedup 1.0000x reference)
import functools

import jax
import jax.numpy as jnp
from jax import lax
from jax.experimental import pallas as pl
from jax.experimental.pallas import tpu as pltpu
from jax.experimental.pallas import tpu_sc as plsc


def _elu(v):
    return jnp.where(v > 0, v, jnp.exp(jnp.minimum(v, 0.0)) - 1.0)


def _mask_pad_rows(x2, n_period):
    BN, F = x2.shape
    NBm = 2000

    def body(x_ref, o_ref):
        i = pl.program_id(0)
        r = i * NBm + lax.broadcasted_iota(jnp.int32, (NBm, 1), 0)
        keep = ((r + 1) % n_period != 0).astype(x_ref.dtype)
        o_ref[...] = x_ref[...] * keep

    return pl.pallas_call(
        body,
        grid=(BN // NBm,),
        in_specs=[pl.BlockSpec((NBm, F), lambda i: (i, 0))],
        out_specs=pl.BlockSpec((NBm, F), lambda i: (i, 0)),
        out_shape=jax.ShapeDtypeStruct((BN, F), x2.dtype),
    )(x2)


def _sc_gather(table, idx_flat, row0, m_rows):
    F = table.shape[1]
    NW = 32
    assert m_rows % (NW * 8) == 0 and row0 % 8 == 0
    m_per_w = m_rows // NW
    CH = next(c for c in (512, 400, 320, 256, 200, 128, 80, 64, 40, 16, 8)
              if m_per_w % c == 0)
    n_ch = m_per_w // CH
    mesh = plsc.VectorSubcoreMesh(core_axis_name="c", subcore_axis_name="s")

    @functools.partial(
        pl.kernel,
        mesh=mesh,
        out_type=jax.ShapeDtypeStruct((m_rows, F), table.dtype),
        scratch_types=[
            pltpu.VMEM((CH,), jnp.int32),
            pltpu.VMEM((CH, F), table.dtype),
            pltpu.SemaphoreType.DMA,
        ],
    )
    def gather_kernel(table_hbm, idx_hbm, out_hbm, idx_v, rows_v, sem):
        wid = lax.axis_index("s") * 2 + lax.axis_index("c")
        base = wid * m_per_w

        @pl.loop(0, n_ch)
        def _(c):
            off = base + c * CH
            pltpu.sync_copy(idx_hbm.at[pl.ds(row0 + off, CH)], idx_v)
            pltpu.async_copy(table_hbm.at[idx_v], rows_v, sem).wait()
            pltpu.sync_copy(rows_v, out_hbm.at[pl.ds(off, CH)])

    return gather_kernel(table, idx_flat)


def _tc_compute(g_c, adjw, x2m, wcr, bc2, wmt, bm2, N, K, FIN, FOUT,
                node_base, row_base, n_nodes, NB):
    nblk = n_nodes // NB

    def body(g_ref, adj_ref, x_ref, wcr_ref, bc_ref, wmt_ref, bm_ref, o_ref):
        i = pl.program_id(0)
        X = g_ref[...].reshape(NB, K, FIN).astype(jnp.bfloat16)
        A = adj_ref[...]
        Y = lax.dot_general(A, X, (((1,), (1,)), ((0,), (0,))),
                            preferred_element_type=jnp.float32
                            ).astype(jnp.bfloat16)
        acc = jnp.zeros((NB, FOUT), jnp.float32)
        for t in range(K):
            acc = acc + jnp.dot(_elu(Y[:, t, :]), wcr_ref[t],
                                preferred_element_type=jnp.float32)
        out_feat = _elu(acc + bc_ref[...])
        nidx = node_base + i * NB + lax.broadcasted_iota(jnp.int32, (NB, 1), 0)
        nmask = (nidx != N - 1).astype(jnp.float32)
        res = jnp.dot(x_ref[...], wmt_ref[...],
                      preferred_element_type=jnp.float32) + bm_ref[...]
        o_ref[...] = out_feat * nmask + res

    return pl.pallas_call(
        body,
        grid=(nblk,),
        in_specs=[
            pl.BlockSpec((NB * K, FIN), lambda i: (i, 0)),
            pl.BlockSpec((NB, K, K), lambda i: (node_base // NB + i, 0, 0)),
            pl.BlockSpec((NB, FIN), lambda i: (row_base // NB + i, 0)),
            pl.BlockSpec((K, FIN, FOUT), lambda i: (0, 0, 0)),
            pl.BlockSpec((1, FOUT), lambda i: (0, 0)),
            pl.BlockSpec((FIN, FOUT), lambda i: (0, 0)),
            pl.BlockSpec((1, FOUT), lambda i: (0, 0)),
        ],
        out_specs=pl.BlockSpec((NB, FOUT), lambda i: (i, 0)),
        out_shape=jax.ShapeDtypeStruct((n_nodes, FOUT), jnp.float32),
        compiler_params=pltpu.CompilerParams(
            dimension_semantics=("arbitrary",)),
    )(g_c, adjw, x2m, wcr, bc2, wmt, bm2)


@jax.jit
def kernel(x, neighbor_index, adjweight, Wc, bc, Wm, bm):
    b, n, fin = x.shape
    k = neighbor_index.shape[-1]
    fout = Wc.shape[0]
    x2 = x.reshape(b * n, fin)
    offs = (jnp.arange(b, dtype=jnp.int32) * n)[:, None, None]
    idx_flat = (neighbor_index.astype(jnp.int32) + offs).reshape(-1)
    x2m = _mask_pad_rows(x2, n)
    wcr = Wc.reshape(fout, k, fin).transpose(1, 2, 0).astype(jnp.bfloat16)
    wmt = Wm.T
    adjb = adjweight.astype(jnp.bfloat16)
    bc2 = bc.reshape(1, -1)
    bm2 = bm.reshape(1, -1)
    CHUNKS_PER_BATCH = 2
    n_nodes = n // CHUNKS_PER_BATCH
    NB = 1000
    n_chunks = b * CHUNKS_PER_BATCH
    m_real = n_nodes * k
    m_pad = -(-m_real // 256) * 256
    pad_vals = jnp.broadcast_to(
        jnp.arange(m_pad - m_real, dtype=jnp.int32)[None, :] % jnp.int32(b * n),
        (n_chunks, m_pad - m_real))
    idx_padded = jnp.concatenate(
        [idx_flat.reshape(n_chunks, m_real), pad_vals], axis=1).reshape(-1)
    outs = []
    for bi in range(b):
        for ci in range(CHUNKS_PER_BATCH):
            node_base = ci * n_nodes
            row_base = bi * n + node_base
            chunk_id = bi * CHUNKS_PER_BATCH + ci
            g_c = _sc_gather(x2m, idx_padded, chunk_id * m_pad, m_pad)
            outs.append(_tc_compute(g_c, adjb, x2m, wcr, bc2, wmt, bm2,
                                    n, k, fin, fout, node_base, row_base,
                                    n_nodes, NB))
    out2 = jnp.concatenate(outs, axis=0)
    return out2.reshape(b, n, fout)


# speedup vs baseline: 1.0005x; 1.0005x over previous
import functools

import jax
import jax.numpy as jnp
from jax import lax
from jax.experimental import pallas as pl
from jax.experimental.pallas import tpu as pltpu
from jax.experimental.pallas import tpu_sc as plsc


def _elu(v):
    return jnp.where(v > 0, v, jnp.exp(jnp.minimum(v, 0.0)) - 1.0)


def _mask_pad_rows(x2, n_period):
    BN, F = x2.shape
    NBm = 2000

    def body(x_ref, o_ref):
        i = pl.program_id(0)
        r = i * NBm + lax.broadcasted_iota(jnp.int32, (NBm, 1), 0)
        keep = ((r + 1) % n_period != 0).astype(x_ref.dtype)
        o_ref[...] = x_ref[...] * keep

    return pl.pallas_call(
        body,
        grid=(BN // NBm,),
        in_specs=[pl.BlockSpec((NBm, F), lambda i: (i, 0))],
        out_specs=pl.BlockSpec((NBm, F), lambda i: (i, 0)),
        out_shape=jax.ShapeDtypeStruct((BN, F), x2.dtype),
    )(x2)


def _sc_gather(table, idx_flat, row0, m_rows):
    F = table.shape[1]
    NW = 32
    assert m_rows % (NW * 8) == 0 and row0 % 8 == 0
    m_per_w = m_rows // NW
    CH = next(c for c in (512, 400, 320, 256, 200, 128, 80, 64, 40, 16, 8)
              if m_per_w % c == 0)
    n_ch = m_per_w // CH
    mesh = plsc.VectorSubcoreMesh(core_axis_name="c", subcore_axis_name="s")

    @functools.partial(
        pl.kernel,
        mesh=mesh,
        out_type=jax.ShapeDtypeStruct((m_rows, F), table.dtype),
        scratch_types=[
            pltpu.VMEM((CH,), jnp.int32),
            pltpu.VMEM((CH, F), table.dtype),
            pltpu.SemaphoreType.DMA,
        ],
    )
    def gather_kernel(table_hbm, idx_hbm, out_hbm, idx_v, rows_v, sem):
        wid = lax.axis_index("s") * 2 + lax.axis_index("c")
        base = wid * m_per_w

        @pl.loop(0, n_ch)
        def _(c):
            off = base + c * CH
            pltpu.sync_copy(idx_hbm.at[pl.ds(row0 + off, CH)], idx_v)
            pltpu.async_copy(table_hbm.at[idx_v], rows_v, sem).wait()
            pltpu.sync_copy(rows_v, out_hbm.at[pl.ds(off, CH)])

    return gather_kernel(table, idx_flat)


def _tc_compute(g_c, adjw, x2m, wcr, bc2, wmt, bm2, N, K, FIN, FOUT,
                node_base, row_base, n_nodes, NB):
    nblk = n_nodes // NB

    def body(g_ref, adj_ref, x_ref, wcr_ref, bc_ref, wmt_ref, bm_ref, o_ref):
        i = pl.program_id(0)
        X = g_ref[...].reshape(NB, K, FIN).astype(jnp.bfloat16)
        A = adj_ref[...]
        Y = lax.dot_general(A, X, (((1,), (1,)), ((0,), (0,))),
                            preferred_element_type=jnp.float32
                            ).astype(jnp.bfloat16)
        acc = jnp.zeros((NB, FOUT), jnp.float32)
        for t in range(K):
            acc = acc + jnp.dot(_elu(Y[:, t, :]), wcr_ref[t],
                                preferred_element_type=jnp.float32)
        out_feat = _elu(acc + bc_ref[...])
        nidx = node_base + i * NB + lax.broadcasted_iota(jnp.int32, (NB, 1), 0)
        nmask = (nidx != N - 1).astype(jnp.float32)
        res = jnp.dot(x_ref[...], wmt_ref[...],
                      preferred_element_type=jnp.float32) + bm_ref[...]
        o_ref[...] = out_feat * nmask + res

    return pl.pallas_call(
        body,
        grid=(nblk,),
        in_specs=[
            pl.BlockSpec((NB * K, FIN), lambda i: (i, 0)),
            pl.BlockSpec((NB, K, K), lambda i: (node_base // NB + i, 0, 0)),
            pl.BlockSpec((NB, FIN), lambda i: (row_base // NB + i, 0)),
            pl.BlockSpec((K, FIN, FOUT), lambda i: (0, 0, 0)),
            pl.BlockSpec((1, FOUT), lambda i: (0, 0)),
            pl.BlockSpec((FIN, FOUT), lambda i: (0, 0)),
            pl.BlockSpec((1, FOUT), lambda i: (0, 0)),
        ],
        out_specs=pl.BlockSpec((NB, FOUT), lambda i: (i, 0)),
        out_shape=jax.ShapeDtypeStruct((n_nodes, FOUT), jnp.float32),
        compiler_params=pltpu.CompilerParams(
            dimension_semantics=("arbitrary",)),
    )(g_c, adjw, x2m, wcr, bc2, wmt, bm2)


@jax.jit
def kernel(x, neighbor_index, adjweight, Wc, bc, Wm, bm):
    b, n, fin = x.shape
    k = neighbor_index.shape[-1]
    fout = Wc.shape[0]
    x2 = x.reshape(b * n, fin)
    offs = (jnp.arange(b, dtype=jnp.int32) * n)[:, None, None]
    idx_flat = (neighbor_index.astype(jnp.int32) + offs).reshape(-1)
    x2m = _mask_pad_rows(x2, n)
    wcr = Wc.reshape(fout, k, fin).transpose(1, 2, 0).astype(jnp.bfloat16)
    wmt = Wm.T
    adjb = adjweight.astype(jnp.bfloat16)
    bc2 = bc.reshape(1, -1)
    bm2 = bm.reshape(1, -1)
    CHUNKS_PER_BATCH = 2
    n_nodes = n // CHUNKS_PER_BATCH
    NB = 1000
    m_real = n_nodes * k
    m_pad = -(-m_real // 256) * 256
    pad_vals = jnp.arange(m_pad - m_real, dtype=jnp.int32) % jnp.int32(b * n)
    outs = []
    for bi in range(b):
        for ci in range(CHUNKS_PER_BATCH):
            node_base = ci * n_nodes
            row_base = bi * n + node_base
            idx_c = lax.dynamic_slice_in_dim(idx_flat, row_base * k, m_real)
            idx_c = jnp.concatenate([idx_c, pad_vals])
            g_c = _sc_gather(x2m, idx_c, 0, m_pad)
            outs.append(_tc_compute(g_c, adjb, x2m, wcr, bc2, wmt, bm2,
                                    n, k, fin, fout, node_base, row_base,
                                    n_nodes, NB))
    out2 = jnp.concatenate(outs, axis=0)
    return out2.reshape(b, n, fout)


# speedup vs baseline: 4.8286x; 4.8264x over previous
import functools

import jax
import jax.numpy as jnp
from jax import lax
from jax.experimental import pallas as pl
from jax.experimental.pallas import tpu as pltpu
from jax.experimental.pallas import tpu_sc as plsc


def _elu(v):
    return jnp.where(v > 0, v, jnp.exp(jnp.minimum(v, 0.0)) - 1.0)


def _mask_pad_rows(x2, n_period):
    BN, F = x2.shape
    NBm = 2000

    def body(x_ref, o_ref):
        i = pl.program_id(0)
        r = i * NBm + lax.broadcasted_iota(jnp.int32, (NBm, 1), 0)
        keep = ((r + 1) % n_period != 0).astype(x_ref.dtype)
        o_ref[...] = x_ref[...] * keep

    return pl.pallas_call(
        body,
        grid=(BN // NBm,),
        in_specs=[pl.BlockSpec((NBm, F), lambda i: (i, 0))],
        out_specs=pl.BlockSpec((NBm, F), lambda i: (i, 0)),
        out_shape=jax.ShapeDtypeStruct((BN, F), x2.dtype),
    )(x2)


def _sc_gather(table, idx_flat, row0, m_rows):
    F = table.shape[1]
    NW = 32
    assert m_rows % (NW * 8) == 0 and row0 % 8 == 0
    m_per_w = m_rows // NW
    CH = next(c for c in (512, 400, 320, 256, 200, 128, 80, 64, 40, 16, 8)
              if m_per_w % c == 0)
    n_ch = m_per_w // CH
    mesh = plsc.VectorSubcoreMesh(core_axis_name="c", subcore_axis_name="s")

    @functools.partial(
        pl.kernel,
        mesh=mesh,
        out_type=jax.ShapeDtypeStruct((m_rows, F), table.dtype),
        scratch_types=[
            pltpu.VMEM((CH,), jnp.int32),
            pltpu.VMEM((CH, F), table.dtype),
            pltpu.SemaphoreType.DMA,
        ],
    )
    def gather_kernel(table_hbm, idx_hbm, out_hbm, idx_v, rows_v, sem):
        wid = lax.axis_index("s") * 2 + lax.axis_index("c")
        base = wid * m_per_w

        @pl.loop(0, n_ch)
        def _(c):
            off = base + c * CH
            pltpu.sync_copy(idx_hbm.at[pl.ds(row0 + off, CH)], idx_v)
            pltpu.async_copy(table_hbm.at[idx_v], rows_v, sem).wait()
            pltpu.sync_copy(rows_v, out_hbm.at[pl.ds(off, CH)])

    return gather_kernel(table, idx_flat)


def _tc_compute(g_c, adjw, x2m, wcr, bc2, wmt, bm2, N, K, FIN, FOUT,
                node_base, row_base, n_nodes, NB):
    nblk = n_nodes // NB

    def body(g_ref, adj_ref, x_ref, wcr_ref, bc_ref, wmt_ref, bm_ref, o_ref):
        i = pl.program_id(0)
        X = g_ref[...].reshape(NB, K, FIN).astype(jnp.bfloat16)
        A = adj_ref[...]
        Y = lax.dot_general(A, X, (((1,), (1,)), ((0,), (0,))),
                            preferred_element_type=jnp.float32
                            ).astype(jnp.bfloat16)
        acc = jnp.zeros((NB, FOUT), jnp.float32)
        for t in range(K):
            acc = acc + jnp.dot(_elu(Y[:, t, :]), wcr_ref[t],
                                preferred_element_type=jnp.float32)
        out_feat = _elu(acc + bc_ref[...])
        nidx = node_base + i * NB + lax.broadcasted_iota(jnp.int32, (NB, 1), 0)
        nmask = (nidx != N - 1).astype(jnp.float32)
        res = jnp.dot(x_ref[...], wmt_ref[...],
                      preferred_element_type=jnp.float32) + bm_ref[...]
        o_ref[...] = out_feat * nmask + res

    return pl.pallas_call(
        body,
        grid=(nblk,),
        in_specs=[
            pl.BlockSpec((NB * K, FIN), lambda i: (i, 0)),
            pl.BlockSpec((NB, K, K), lambda i: (node_base // NB + i, 0, 0)),
            pl.BlockSpec((NB, FIN), lambda i: (row_base // NB + i, 0)),
            pl.BlockSpec((K, FIN, FOUT), lambda i: (0, 0, 0)),
            pl.BlockSpec((1, FOUT), lambda i: (0, 0)),
            pl.BlockSpec((FIN, FOUT), lambda i: (0, 0)),
            pl.BlockSpec((1, FOUT), lambda i: (0, 0)),
        ],
        out_specs=pl.BlockSpec((NB, FOUT), lambda i: (i, 0)),
        out_shape=jax.ShapeDtypeStruct((n_nodes, FOUT), jnp.float32),
        compiler_params=pltpu.CompilerParams(
            dimension_semantics=("arbitrary",)),
    )(g_c, adjw, x2m, wcr, bc2, wmt, bm2)


@jax.jit
def kernel(x, neighbor_index, adjweight, Wc, bc, Wm, bm):
    b, n, fin = x.shape
    k = neighbor_index.shape[-1]
    fout = Wc.shape[0]
    x2 = x.reshape(b * n, fin)
    offs = (jnp.arange(b, dtype=jnp.int32) * n)[:, None, None]
    idx_flat = (neighbor_index.astype(jnp.int32) + offs).reshape(-1)
    x2m = _mask_pad_rows(x2, n)
    wcr = Wc.reshape(fout, k, fin).transpose(1, 2, 0).astype(jnp.bfloat16)
    wmt = Wm.T
    adjb = adjweight.astype(jnp.bfloat16)
    bc2 = bc.reshape(1, -1)
    bm2 = bm.reshape(1, -1)
    CHUNKS_PER_BATCH = 2
    n_nodes = n // CHUNKS_PER_BATCH
    NB = 1000
    m_real = n_nodes * k
    m_pad = -(-m_real // (32 * 512)) * (32 * 512)
    pad_vals = jnp.arange(m_pad - m_real, dtype=jnp.int32) % jnp.int32(b * n)
    outs = []
    for bi in range(b):
        for ci in range(CHUNKS_PER_BATCH):
            node_base = ci * n_nodes
            row_base = bi * n + node_base
            idx_c = lax.dynamic_slice_in_dim(idx_flat, row_base * k, m_real)
            idx_c = jnp.concatenate([idx_c, pad_vals])
            g_c = _sc_gather(x2m, idx_c, 0, m_pad)
            outs.append(_tc_compute(g_c, adjb, x2m, wcr, bc2, wmt, bm2,
                                    n, k, fin, fout, node_base, row_base,
                                    n_nodes, NB))
    out2 = jnp.concatenate(outs, axis=0)
    return out2.reshape(b, n, fout)
